# baseline (speedup 1.0000x reference)
import jax
import jax.numpy as jnp
from jax import lax
from jax.experimental import pallas as pl
from jax.experimental.pallas import tpu as pltpu
from jax.experimental.pallas import tpu_sc as plsc

B = 16384
D = 64
NC, NS = 2, 16
NW = NC * NS
BPW = B // NW
CH = 128
NCH = BPW // CH
L = 16


def _body(x_hbm, uid_hbm, iid_hbm, out_hbm,
          x_v, idx_u, idx_i, rows_u, rows_i, out_v, sem):
    wid = lax.axis_index("s") * NC + lax.axis_index("c")
    base = wid * BPW

    pltpu.sync_copy(x_hbm.at[pl.ds(base, BPW)], x_v)
    lanes = lax.iota(jnp.int32, L)
    zero = jnp.zeros((L,), jnp.int32)
    one = jnp.full((L,), 1, jnp.int32)
    for c in range(NCH):
        for k in range(CH // L):
            rows16 = jnp.full((L,), c * CH + k * L, jnp.int32) + lanes
            idx_u[c, pl.ds(k * L, L)] = plsc.load_gather(x_v, [rows16, zero])
            idx_i[c, pl.ds(k * L, L)] = plsc.load_gather(x_v, [rows16, one])

    copies = []
    for c in range(NCH):
        copies.append(pltpu.async_copy(
            uid_hbm.at[idx_u.at[c]], rows_u.at[pl.ds(c * CH, CH)], sem))
        copies.append(pltpu.async_copy(
            iid_hbm.at[idx_i.at[c]], rows_i.at[pl.ds(c * CH, CH)], sem))
    for cp in copies:
        cp.wait()

    def blk(b, carry):
        r0 = b * L
        row_idx = r0 + lanes
        accs = [jnp.zeros((L,), jnp.float32) for _ in range(4)]
        for d in range(D):
            col = jnp.full((L,), d, jnp.int32)
            u = plsc.load_gather(rows_u, [row_idx, col])
            v = plsc.load_gather(rows_i, [row_idx, col])
            accs[d % 4] = accs[d % 4] + u * v
        out_v[pl.ds(r0, L)] = (accs[0] + accs[1]) + (accs[2] + accs[3])
        return carry

    lax.fori_loop(0, BPW // L, blk, 0)
    pltpu.sync_copy(out_v, out_hbm.at[pl.ds(base, BPW)])


def kernel(x, uid_table, iid_table):
    mesh = plsc.VectorSubcoreMesh(
        core_axis_name="c", subcore_axis_name="s",
        num_cores=NC, num_subcores=NS)
    run = pl.kernel(
        _body,
        out_type=jax.ShapeDtypeStruct((B,), jnp.float32),
        mesh=mesh,
        compiler_params=pltpu.CompilerParams(
            needs_layout_passes=False, use_tc_tiling_on_sc=False),
        scratch_types=[
            pltpu.VMEM((BPW, 2), jnp.int32),
            pltpu.VMEM((NCH, CH), jnp.int32),
            pltpu.VMEM((NCH, CH), jnp.int32),
            pltpu.VMEM((BPW, D), jnp.float32),
            pltpu.VMEM((BPW, D), jnp.float32),
            pltpu.VMEM((BPW,), jnp.float32),
            pltpu.SemaphoreType.DMA,
        ],
    )
    return run(x, uid_table, iid_table)


# speedup vs baseline: 1.1960x; 1.1960x over previous
import jax
import jax.numpy as jnp
from jax import lax
from jax.experimental import pallas as pl
from jax.experimental.pallas import tpu as pltpu
from jax.experimental.pallas import tpu_sc as plsc

B = 16384
D = 64
NC, NS = 2, 16
NW = NC * NS
BPW = B // NW
CH = 32
NCH = BPW // CH
L = 16
G = 8


def _body(ux_hbm, ix_hbm, uid_hbm, iid_hbm, out_hbm,
          uxv, ixv, buf_u, buf_i, out_v, sem):
    wid = lax.axis_index("s") * NC + lax.axis_index("c")
    base = wid * BPW

    pltpu.sync_copy(ux_hbm.at[pl.ds(base, BPW)], uxv)
    pltpu.sync_copy(ix_hbm.at[pl.ds(base, BPW)], ixv)

    lanes = lax.iota(jnp.int32, L)
    perms = [lanes ^ sh for sh in (8, 4, 2, 1)]
    u3 = uid_hbm.reshape(100000 // G, G, D)

    def chunk(c, carry):
        cb = c * CH
        for h in range(CH // L):
            tq = pl.ds(cb + h * L, L)
            tu16 = uxv[tq] & ~7
            ti16 = ixv[tq] & ~7
            for jj in range(L):
                jc = h * L + jj
                tu = pl.multiple_of(tu16[jj], G)
                ti = pl.multiple_of(ti16[jj], G)
                pltpu.async_copy(uid_hbm.at[pl.ds(tu, G)], buf_u.at[jc], sem)
                pltpu.async_copy(iid_hbm.at[pl.ds(ti, G)], buf_i.at[jc], sem)
        pltpu.make_async_copy(u3.at[pl.ds(0, CH)], buf_u, sem).wait()
        pltpu.make_async_copy(u3.at[pl.ds(0, CH)], buf_i, sem).wait()

        for g in range(CH // L):
            r0 = g * L
            sq = pl.ds(cb + r0, L)
            su16 = uxv[sq] & 7
            si16 = ixv[sq] & 7
            out16 = jnp.zeros((L,), jnp.float32)
            for jj in range(L):
                jc = r0 + jj
                su = su16[jj]
                si = si16[jj]
                acc = None
                for k in range(D // L):
                    u = buf_u[jc, su, pl.ds(k * L, L)]
                    v = buf_i[jc, si, pl.ds(k * L, L)]
                    p = u * v
                    acc = p if acc is None else acc + p
                for p_ in perms:
                    acc = acc + jnp.take_along_axis(
                        acc, p_, axis=0, mode="promise_in_bounds")
                out16 = jnp.where(lanes == jj, acc, out16)
            out_v[pl.ds(cb + r0, L)] = out16
        return carry

    lax.fori_loop(0, NCH, chunk, 0)
    pltpu.sync_copy(out_v, out_hbm.at[pl.ds(base, BPW)])


def kernel(x, uid_table, iid_table):
    ux = x[:, 0]
    ix = x[:, 1]
    mesh = plsc.VectorSubcoreMesh(
        core_axis_name="c", subcore_axis_name="s",
        num_cores=NC, num_subcores=NS)
    run = pl.kernel(
        _body,
        out_type=jax.ShapeDtypeStruct((B,), jnp.float32),
        mesh=mesh,
        compiler_params=pltpu.CompilerParams(
            needs_layout_passes=False, use_tc_tiling_on_sc=True),
        scratch_types=[
            pltpu.VMEM((BPW,), jnp.int32),
            pltpu.VMEM((BPW,), jnp.int32),
            pltpu.VMEM((CH, G, D), jnp.float32),
            pltpu.VMEM((CH, G, D), jnp.float32),
            pltpu.VMEM((BPW,), jnp.float32),
            pltpu.SemaphoreType.DMA,
        ],
    )
    return run(ux, ix, uid_table, iid_table)


# speedup vs baseline: 1.2979x; 1.0853x over previous
import jax
import jax.numpy as jnp
from jax import lax
from jax.experimental import pallas as pl
from jax.experimental.pallas import tpu as pltpu
from jax.experimental.pallas import tpu_sc as plsc

B = 16384
D = 64
NC, NS = 2, 16
NW = NC * NS
BPW = B // NW
CH = 16
NCH = BPW // CH
L = 16
G = 8


def _body(ux_hbm, ix_hbm, uid_hbm, iid_hbm, out_hbm,
          uxv, ixv, bu0, bi0, bu1, bi1, out_v, sem):
    wid = lax.axis_index("s") * NC + lax.axis_index("c")
    base = wid * BPW

    pltpu.sync_copy(ux_hbm.at[pl.ds(base, BPW)], uxv)
    pltpu.sync_copy(ix_hbm.at[pl.ds(base, BPW)], ixv)

    lanes = lax.iota(jnp.int32, L)
    perms = [lanes ^ sh for sh in (8, 4, 2, 1)]
    u3 = uid_hbm.reshape(100000 // G, G, D)

    def fire(c, bu, bi):
        cb = c * CH
        for h in range(CH // L):
            tq = pl.ds(cb + h * L, L)
            tu16 = uxv[tq] & ~7
            ti16 = ixv[tq] & ~7
            for jj in range(L):
                jc = h * L + jj
                tu = pl.multiple_of(tu16[jj], G)
                ti = pl.multiple_of(ti16[jj], G)
                pltpu.async_copy(uid_hbm.at[pl.ds(tu, G)], bu.at[jc], sem)
                pltpu.async_copy(iid_hbm.at[pl.ds(ti, G)], bi.at[jc], sem)

    def drain(bu, bi):
        pltpu.make_async_copy(u3.at[pl.ds(0, CH)], bu, sem).wait()
        pltpu.make_async_copy(u3.at[pl.ds(0, CH)], bi, sem).wait()

    def compute(c, bu, bi):
        cb = c * CH
        for g in range(CH // L):
            r0 = g * L
            sq = pl.ds(cb + r0, L)
            su16 = uxv[sq] & 7
            si16 = ixv[sq] & 7
            out16 = jnp.zeros((L,), jnp.float32)
            for jj in range(L):
                jc = r0 + jj
                su = su16[jj]
                si = si16[jj]
                acc = None
                for k in range(D // L):
                    u = bu[jc, su, pl.ds(k * L, L)]
                    v = bi[jc, si, pl.ds(k * L, L)]
                    p = u * v
                    acc = p if acc is None else acc + p
                for p_ in perms:
                    acc = acc + jnp.take_along_axis(
                        acc, p_, axis=0, mode="promise_in_bounds")
                out16 = jnp.where(lanes == jj, acc, out16)
            out_v[pl.ds(cb + r0, L)] = out16

    fire(0, bu0, bi0)

    def pair(i, carry):
        c0 = i * 2
        c1 = c0 + 1
        fire(c1, bu1, bi1)
        drain(bu0, bi0)
        compute(c0, bu0, bi0)
        c2 = jnp.minimum(c0 + 2, NCH - 1)
        fire(c2, bu0, bi0)
        drain(bu1, bi1)
        compute(c1, bu1, bi1)
        return carry

    lax.fori_loop(0, NCH // 2, pair, 0)
    drain(bu0, bi0)
    pltpu.sync_copy(out_v, out_hbm.at[pl.ds(base, BPW)])


def kernel(x, uid_table, iid_table):
    ux = x[:, 0]
    ix = x[:, 1]
    mesh = plsc.VectorSubcoreMesh(
        core_axis_name="c", subcore_axis_name="s",
        num_cores=NC, num_subcores=NS)
    run = pl.kernel(
        _body,
        out_type=jax.ShapeDtypeStruct((B,), jnp.float32),
        mesh=mesh,
        compiler_params=pltpu.CompilerParams(
            needs_layout_passes=False, use_tc_tiling_on_sc=True),
        scratch_types=[
            pltpu.VMEM((BPW,), jnp.int32),
            pltpu.VMEM((BPW,), jnp.int32),
            pltpu.VMEM((CH, G, D), jnp.float32),
            pltpu.VMEM((CH, G, D), jnp.float32),
            pltpu.VMEM((CH, G, D), jnp.float32),
            pltpu.VMEM((CH, G, D), jnp.float32),
            pltpu.VMEM((BPW,), jnp.float32),
            pltpu.SemaphoreType.DMA,
        ],
    )
    return run(ux, ix, uid_table, iid_table)
